# baseline (speedup 1.0000x reference)
import functools
import jax
import jax.numpy as jnp
from jax import lax
from jax.experimental import pallas as pl
from jax.experimental.pallas import tpu as pltpu
from jax.experimental.pallas import tpu_sc as plsc

CLS = 1000
N0 = 4096
N1 = 26
NW = 32
PPW = N0 // NW


def _sc_body(x_hbm, out_hbm, xv, buf):
    wid = lax.axis_index("s") * 2 + lax.axis_index("c")
    base = wid * PPW

    zeros16f = jnp.zeros((16,), jnp.float32)
    iota16 = lax.iota(jnp.int32, 16)

    def zero_plane(r, _):
        def zb(i, _):
            buf[0, r, pl.ds(pl.multiple_of(i * 16, 16), 16)] = zeros16f
            return 0
        lax.fori_loop(0, 63, zb, 0)
        return 0

    lax.fori_loop(0, N1, zero_plane, 0)

    def chunk_body(k, _):
        i0 = base + k
        pltpu.sync_copy(x_hbm.at[pl.ds(i0 * 32, 32)], xv)
        xva = xv[pl.ds(0, 16)]
        xvb = xv[pl.ds(16, 16)]

        for r in range(N1):
            xs = xva[r] if r < 16 else xvb[r - 16]
            start = pl.multiple_of((xs >> 4) << 4, 16)
            w = buf[0, r, pl.ds(start, 16)]
            buf[0, r, pl.ds(start, 16)] = jnp.where(
                iota16 + start == xs, 1.0, w)

        pltpu.sync_copy(buf, out_hbm.at[pl.ds(i0, 1)])

        for r in range(N1):
            xs = xva[r] if r < 16 else xvb[r - 16]
            start = pl.multiple_of((xs >> 4) << 4, 16)
            buf[0, r, pl.ds(start, 16)] = zeros16f
        return 0

    lax.fori_loop(0, PPW, chunk_body, 0)


def kernel(x):
    xpad = jnp.pad(x, ((0, 0), (0, 6)))
    mesh = plsc.VectorSubcoreMesh(core_axis_name="c", subcore_axis_name="s")
    f = pl.kernel(
        _sc_body,
        out_type=jax.ShapeDtypeStruct((N0, N1, CLS), jnp.float32),
        mesh=mesh,
        scratch_types=[
            pltpu.VMEM((32,), jnp.int32),
            pltpu.VMEM((1, N1, CLS), jnp.float32),
        ],
    )
    return f(xpad.reshape(N0 * 32))


# speedup vs baseline: 5.5273x; 5.5273x over previous
import jax
import jax.numpy as jnp
from jax.experimental import pallas as pl
from jax.experimental.pallas import tpu as pltpu

CLS = 1000
N0 = 4096
N1 = 26
BC = 200
NCB = CLS // BC


def _onehot_body(x_ref, o_ref):
    i1 = pl.program_id(0)
    jc = pl.program_id(1)
    xrow = x_ref[pl.ds(i1, 1), :]
    ci = jax.lax.broadcasted_iota(jnp.int32, (BC, N0), 0) + jc * BC
    o_ref[0] = (ci == xrow).astype(jnp.float32)


def kernel(x):
    xt = x.T
    out_t = pl.pallas_call(
        _onehot_body,
        grid=(N1, NCB),
        in_specs=[pl.BlockSpec((N1, N0), lambda i, j: (0, 0))],
        out_specs=pl.BlockSpec((1, BC, N0), lambda i, j: (i, j, 0)),
        out_shape=jax.ShapeDtypeStruct((N1, CLS, N0), jnp.float32),
        compiler_params=pltpu.CompilerParams(
            dimension_semantics=("arbitrary", "arbitrary"),
        ),
    )(xt)
    return jnp.transpose(out_t, (2, 0, 1))
